# baseline (speedup 1.0000x reference)
import jax
import jax.numpy as jnp
from jax import lax
from jax.experimental import pallas as pl

NPTS = 500
PAD = 512
CF = 160
DIM = 64


def _fused_body(f_ref, f1_ref, f2_ref, dff_ref, dfs_ref, comb_ref,
                w1_ref, b1_ref, w3_ref, b3_ref, out_ref):
    iota = lax.broadcasted_iota(jnp.int32, (PAD, PAD), 1)
    cdims = (((1,), (1,)), ((), ()))
    padp = PAD - NPTS

    def padc(x):
        return jnp.pad(x, ((0, 0), (0, padp)))

    def padr(x):
        return jnp.pad(x, ((0, padp), (0, 0)))

    comb = comb_ref[...]
    inds1 = padr(comb[0:NPTS])
    inds2 = padr(comb[NPTS:2 * NPTS])
    wei1 = padr(lax.bitcast_convert_type(comb[2 * NPTS:3 * NPTS],
                                         jnp.float32)) * 0.125
    inds = padr(comb[3 * NPTS:4 * NPTS, 0:4])

    at = jnp.zeros((PAD, PAD), jnp.float32)
    dt = jnp.zeros((PAD, PAD), jnp.float32)
    ct = jnp.zeros((PAD, PAD), jnp.float32)
    bt = jnp.zeros((PAD, PAD), jnp.float32)
    for j in range(8):
        mask = inds1[:, j:j + 1] == iota
        at = at + jnp.where(mask, wei1[:, j:j + 1], 0.0)
        if j < 4:
            dt = dt + jnp.where(mask, 0.25, 0.0)
            bt = bt + jnp.where(inds[:, j:j + 1] == iota, 0.25, 0.0)
        ct = ct + jnp.where(inds2[:, j:j + 1] == iota, 0.125, 0.0)

    s1row = lax.dot_general(jnp.ones((1, PAD), jnp.float32), at, cdims,
                            preferred_element_type=jnp.float32)

    f = padc(f_ref[...])
    fs1 = f * s1row
    pix = fs1 - lax.dot_general(padc(f1_ref[...]), at, cdims,
                                preferred_element_type=jnp.float32)
    pt = fs1 - lax.dot_general(padc(f2_ref[...]), at, cdims,
                               preferred_element_type=jnp.float32)
    plus = pix + lax.dot_general(pt, bt, cdims,
                                 preferred_element_type=jnp.float32)
    ds = jax.nn.sigmoid(
        jnp.dot(w1_ref[...], plus, preferred_element_type=jnp.float32)
        + b1_ref[...][:, None])
    m = lax.dot_general(ds, ct, cdims, preferred_element_type=jnp.float32)
    new_f = padc(dff_ref[...]) * m
    plus2 = padc(dfs_ref[...]) + lax.dot_general(
        new_f, dt, cdims, preferred_element_type=jnp.float32)
    res = (jnp.dot(w3_ref[...], plus2, preferred_element_type=jnp.float32)
           + b3_ref[...][:, None])
    out_ref[...] = res[None, :, :NPTS]


@jax.jit
def kernel(feat, feat1, feat2, inds, inds1, inds2, wei1, wei2,
           dens_feat_f, dens_feat_s, W1, b1, W3, b3):
    del wei2
    comb = jnp.concatenate([
        inds1[0].astype(jnp.int32).reshape(NPTS, 8),
        inds2[0].astype(jnp.int32).reshape(NPTS, 8),
        lax.bitcast_convert_type(wei1[0], jnp.int32).reshape(NPTS, 8),
        jnp.pad(inds[0].astype(jnp.int32).reshape(NPTS, 4), ((0, 0), (0, 4))),
    ], axis=0)
    return pl.pallas_call(
        _fused_body,
        out_shape=jax.ShapeDtypeStruct((1, CF, NPTS), jnp.float32),
    )(feat[0], feat1[0], feat2[0],
      dens_feat_f[0], dens_feat_s[0], comb,
      W1, b1, W3, b3)


# speedup vs baseline: 1.1729x; 1.1729x over previous
import jax
import jax.numpy as jnp
import numpy as np
from jax import lax
from jax.experimental import pallas as pl

NPTS = 500
PAD = 512
CF = 160
DIM = 64
FLAT = 4096

_P8 = np.zeros((FLAT, PAD), np.float32)
_P8[np.arange(PAD) * 8, np.arange(PAD)] = 1.0
_P4 = np.zeros((FLAT, PAD), np.float32)
_P4[np.arange(PAD) * 4, np.arange(PAD)] = 1.0


def _fused_body(f_ref, f1_ref, f2_ref, dff_ref, dfs_ref, stk_ref,
                p8_ref, p4_ref, w1_ref, b1_ref, w3_ref, b3_ref, out_ref):
    iota = lax.broadcasted_iota(jnp.int32, (PAD, PAD), 0)
    padp = PAD - NPTS

    def padc(x):
        return jnp.pad(x, ((0, 0), (0, padp)))

    stk = stk_ref[...]
    i1r = stk[0:1].astype(jnp.float32)
    i2r = stk[1:2].astype(jnp.float32)
    wbr = lax.bitcast_convert_type(stk[2:3], jnp.float32)
    i4r = stk[3:4].astype(jnp.float32)

    def _roll(x, j):
        return x if j == 0 else jnp.roll(x, -j, axis=1)

    sel8 = jnp.concatenate(
        [_roll(i1r, j) for j in range(8)]
        + [_roll(i2r, j) for j in range(8)]
        + [_roll(wbr, j) for j in range(8)], axis=0)
    ext8 = jnp.dot(sel8, p8_ref[...], preferred_element_type=jnp.float32)
    sel4 = jnp.concatenate(
        [_roll(i4r, j) for j in range(4)], axis=0)
    ext4 = jnp.dot(sel4, p4_ref[...], preferred_element_type=jnp.float32)

    def irow(x):
        return (x + 0.5).astype(jnp.int32)

    at = jnp.zeros((PAD, PAD), jnp.float32)
    dt = jnp.zeros((PAD, PAD), jnp.float32)
    ct = jnp.zeros((PAD, PAD), jnp.float32)
    bt = jnp.zeros((PAD, PAD), jnp.float32)
    s1row = jnp.zeros((1, PAD), jnp.float32)
    for j in range(8):
        mask = irow(ext8[j:j + 1]) == iota
        wj = ext8[16 + j:17 + j] * 0.125
        s1row = s1row + wj
        at = at + jnp.where(mask, wj, 0.0)
        if j < 4:
            dt = dt + jnp.where(mask, 0.25, 0.0)
            bt = bt + jnp.where(irow(ext4[j:j + 1]) == iota, 0.25, 0.0)
        ct = ct + jnp.where(irow(ext8[8 + j:9 + j]) == iota, 0.125, 0.0)

    f = padc(f_ref[...])
    fs1 = f * s1row
    pix = fs1 - jnp.dot(padc(f1_ref[...]), at,
                        preferred_element_type=jnp.float32)
    pt = fs1 - jnp.dot(padc(f2_ref[...]), at,
                       preferred_element_type=jnp.float32)
    plus = pix + jnp.dot(pt, bt, preferred_element_type=jnp.float32)
    ds = jax.nn.sigmoid(
        jnp.dot(w1_ref[...], plus, preferred_element_type=jnp.float32)
        + b1_ref[...][:, None])
    m = jnp.dot(ds, ct, preferred_element_type=jnp.float32)
    new_f = padc(dff_ref[...]) * m
    plus2 = padc(dfs_ref[...]) + jnp.dot(new_f, dt,
                                         preferred_element_type=jnp.float32)
    res = (jnp.dot(w3_ref[...], plus2, preferred_element_type=jnp.float32)
           + b3_ref[...][:, None])
    out_ref[...] = res[None, :, :NPTS]


@jax.jit
def kernel(feat, feat1, feat2, inds, inds1, inds2, wei1, wei2,
           dens_feat_f, dens_feat_s, W1, b1, W3, b3):
    del wei2
    zpad = ((0, 0), (0, FLAT - 4000))
    stk = jnp.concatenate([
        jnp.pad(inds1.astype(jnp.int32), zpad),
        jnp.pad(inds2.astype(jnp.int32), zpad),
        jnp.pad(lax.bitcast_convert_type(wei1, jnp.int32), zpad),
        jnp.pad(inds.astype(jnp.int32), ((0, 0), (0, FLAT - 2000))),
    ], axis=0)
    return pl.pallas_call(
        _fused_body,
        out_shape=jax.ShapeDtypeStruct((1, CF, NPTS), jnp.float32),
    )(feat[0], feat1[0], feat2[0],
      dens_feat_f[0], dens_feat_s[0], stk,
      jnp.asarray(_P8), jnp.asarray(_P4),
      W1, b1, W3, b3)


# speedup vs baseline: 1.3811x; 1.1775x over previous
import jax
import jax.numpy as jnp
from jax import lax
from jax.experimental import pallas as pl

NPTS = 500
PAD = 512
CF = 160
DIM = 64


def _fused_body(f_ref, f1_ref, f2_ref, dff_ref, dfs_ref, comb_ref,
                w1_ref, b1_ref, w3_ref, b3_ref, out_ref):
    iota = lax.broadcasted_iota(jnp.int32, (PAD, PAD), 0)
    padp = PAD - NPTS

    def padc(x):
        return jnp.pad(x, ((0, 0), (0, padp)))

    comb = comb_ref[...]

    def jrow(j, off, fill):
        return jnp.pad(comb[j:j + 1, off:off + NPTS], ((0, 0), (0, padp)),
                       constant_values=fill)

    at = jnp.zeros((PAD, PAD), jnp.float32)
    dt = jnp.zeros((PAD, PAD), jnp.float32)
    ct = jnp.zeros((PAD, PAD), jnp.float32)
    bt = jnp.zeros((PAD, PAD), jnp.float32)
    s1row = jnp.zeros((1, PAD), jnp.float32)
    for j in range(8):
        mask = jrow(j, 0, -1) == iota
        wj = lax.bitcast_convert_type(jrow(j, 1000, 0), jnp.float32) * 0.125
        s1row = s1row + wj
        at = at + jnp.where(mask, wj, 0.0)
        if j < 4:
            dt = dt + jnp.where(mask, 0.25, 0.0)
            bt = bt + jnp.where(jrow(j, 1500, -1) == iota, 0.25, 0.0)
        ct = ct + jnp.where(jrow(j, 500, -1) == iota, 0.125, 0.0)

    f = padc(f_ref[...])
    fs1 = f * s1row
    pix = fs1 - jnp.dot(padc(f1_ref[...]), at,
                        preferred_element_type=jnp.float32)
    pt = fs1 - jnp.dot(padc(f2_ref[...]), at,
                       preferred_element_type=jnp.float32)
    plus = pix + jnp.dot(pt, bt, preferred_element_type=jnp.float32)
    ds = jax.nn.sigmoid(
        jnp.dot(w1_ref[...], plus, preferred_element_type=jnp.float32)
        + b1_ref[...][:, None])
    m = jnp.dot(ds, ct, preferred_element_type=jnp.float32)
    new_f = padc(dff_ref[...]) * m
    plus2 = padc(dfs_ref[...]) + jnp.dot(new_f, dt,
                                         preferred_element_type=jnp.float32)
    res = (jnp.dot(w3_ref[...], plus2, preferred_element_type=jnp.float32)
           + b3_ref[...][:, None])
    out_ref[...] = res[None, :, :NPTS]


@jax.jit
def kernel(feat, feat1, feat2, inds, inds1, inds2, wei1, wei2,
           dens_feat_f, dens_feat_s, W1, b1, W3, b3):
    del wei2
    comb = jnp.concatenate([
        inds1[0].astype(jnp.int32).reshape(NPTS, 8).T,
        inds2[0].astype(jnp.int32).reshape(NPTS, 8).T,
        lax.bitcast_convert_type(wei1[0], jnp.int32).reshape(NPTS, 8).T,
        jnp.pad(inds[0].astype(jnp.int32).reshape(NPTS, 4).T,
                ((0, 4), (0, 0))),
    ], axis=1)
    return pl.pallas_call(
        _fused_body,
        out_shape=jax.ShapeDtypeStruct((1, CF, NPTS), jnp.float32),
    )(feat[0], feat1[0], feat2[0],
      dens_feat_f[0], dens_feat_s[0], comb,
      W1, b1, W3, b3)
